# Initial kernel scaffold; baseline (speedup 1.0000x reference)
import jax
import jax.numpy as jnp
from jax.experimental import pallas as pl


def kernel(q_nodes, k_nodes, hyperedges, Wq, bq, Wk, bk, Wv, bv, W1, b1, W2, b2, ln1_g, ln1_b, ln2_g, ln2_b):
    raise NotImplementedError("write your pallas kernel here")


# speedup vs baseline: 5.0341x; 5.0341x over previous
import functools
import math

import jax
import jax.numpy as jnp
from jax import lax
from jax.experimental import pallas as pl
from jax.experimental.pallas import tpu as pltpu
from jax.experimental.pallas import tpu_sc as plsc

NHEADS = 8


def _pack_halves(y, H):
    lo = lax.bitcast_convert_type(y[:, :H // 2].astype(jnp.bfloat16), jnp.uint16)
    hi = lax.bitcast_convert_type(y[:, H // 2:].astype(jnp.bfloat16), jnp.uint16)
    w = lo.astype(jnp.uint32) | (hi.astype(jnp.uint32) << 16)
    return lax.bitcast_convert_type(w, jnp.int32)


def _proj_q(x, wt, b):
    N, H = x.shape

    def kfn(x_ref, w_ref, b_ref, o_ref):
        y = (jnp.dot(x_ref[...].astype(jnp.bfloat16), w_ref[...],
                     preferred_element_type=jnp.float32)
             + b_ref[...])
        o_ref[...] = _pack_halves(y, H)

    blk = 512
    return pl.pallas_call(
        kfn,
        grid=(N // blk,),
        in_specs=[
            pl.BlockSpec((blk, H), lambda i: (i, 0)),
            pl.BlockSpec((H, H), lambda i: (0, 0)),
            pl.BlockSpec((1, H), lambda i: (0, 0)),
        ],
        out_specs=pl.BlockSpec((blk, H // 2), lambda i: (i, 0)),
        out_shape=jax.ShapeDtypeStruct((N, H // 2), jnp.int32),
    )(x, wt, b.reshape(1, H))


def _proj_kv(x, wkt, bk, wvt, bv):
    N, H = x.shape

    def kfn(x_ref, wk_ref, bk_ref, wv_ref, bv_ref, ok_ref, ov_ref):
        xv = x_ref[...].astype(jnp.bfloat16)
        yk = (jnp.dot(xv, wk_ref[...], preferred_element_type=jnp.float32)
              + bk_ref[...])
        yv = (jnp.dot(xv, wv_ref[...], preferred_element_type=jnp.float32)
              + bv_ref[...])
        ok_ref[...] = _pack_halves(yk, H)
        ov_ref[...] = _pack_halves(yv, H)

    blk = 512
    return pl.pallas_call(
        kfn,
        grid=(N // blk,),
        in_specs=[
            pl.BlockSpec((blk, H), lambda i: (i, 0)),
            pl.BlockSpec((H, H), lambda i: (0, 0)),
            pl.BlockSpec((1, H), lambda i: (0, 0)),
            pl.BlockSpec((H, H), lambda i: (0, 0)),
            pl.BlockSpec((1, H), lambda i: (0, 0)),
        ],
        out_specs=[
            pl.BlockSpec((blk, H // 2), lambda i: (i, 0)),
            pl.BlockSpec((blk, H // 2), lambda i: (i, 0)),
        ],
        out_shape=[
            jax.ShapeDtypeStruct((N, H // 2), jnp.int32),
            jax.ShapeDtypeStruct((N, H // 2), jnp.int32),
        ],
    )(x, wkt, bk.reshape(1, H), wvt, bv.reshape(1, H))


def _post(q, upd, w1t, b1, w2t, b2, g1, bb1, g2, bb2):
    N, H = q.shape

    def _ln(x, g, b):
        mu = jnp.mean(x, axis=-1, keepdims=True)
        var = jnp.var(x, axis=-1, keepdims=True)
        return (x - mu) / jnp.sqrt(var + 1e-5) * g + b

    def kfn(q_ref, u_ref, w1_ref, b1_ref, w2_ref, b2_ref,
            g1_ref, bb1_ref, g2_ref, bb2_ref, o_ref):
        x = _ln(q_ref[...] + u_ref[...], g1_ref[...], bb1_ref[...])
        f = jnp.maximum(
            jnp.dot(x.astype(jnp.bfloat16), w1_ref[...],
                    preferred_element_type=jnp.float32)
            + b1_ref[...], 0.0)
        f = (jnp.dot(f.astype(jnp.bfloat16), w2_ref[...],
                     preferred_element_type=jnp.float32)
             + b2_ref[...])
        o_ref[...] = _ln(x + f, g2_ref[...], bb2_ref[...])

    blk = 512
    full = pl.BlockSpec((H, H), lambda i: (0, 0))
    vec = pl.BlockSpec((1, H), lambda i: (0, 0))
    row = pl.BlockSpec((blk, H), lambda i: (i, 0))
    return pl.pallas_call(
        kfn,
        grid=(N // blk,),
        in_specs=[row, row, full, vec, full, vec, vec, vec, vec, vec],
        out_specs=row,
        out_shape=jax.ShapeDtypeStruct((N, H), jnp.float32),
    )(q, upd, w1t, b1.reshape(1, H), w2t, b2.reshape(1, H),
      g1.reshape(1, H), bb1.reshape(1, H), g2.reshape(1, H), bb2.reshape(1, H))


def _build_routing(qi, ki, B, Q, K, E, M):
    nb = 16
    rpb = Q // nb
    S = E + nb * 16
    i32 = jnp.int32
    bucket = qi // rpb
    perm = jnp.argsort(bucket, axis=1, stable=True)
    qi_s = jnp.take_along_axis(qi, perm, axis=1)
    ki_s = jnp.take_along_axis(ki, perm[:, :, None], axis=1)
    sb = jnp.take_along_axis(bucket, perm, axis=1)
    cnt = (bucket[:, :, None] == jnp.arange(nb)[None, None, :]).sum(1)
    cnt = cnt.astype(i32)
    pc = ((cnt + 15) // 16) * 16
    z = jnp.zeros((B, 1), i32)
    aoff = jnp.concatenate([z, jnp.cumsum(pc, axis=1)], axis=1)
    coff = jnp.concatenate([z, jnp.cumsum(cnt, axis=1)], axis=1)
    rank = jnp.arange(E, dtype=i32)[None, :] - jnp.take_along_axis(coff, sb, axis=1)
    pos = jnp.take_along_axis(aoff, sb, axis=1) + rank
    bix = jnp.arange(B, dtype=i32)[:, None]
    boq = bix * Q
    bok = bix[:, :, None] * K
    qg = jnp.zeros((B, S), i32).at[bix, pos].set(qi_s + boq)
    qsl = jnp.full((B, S), rpb, i32).at[bix, pos].set(qi_s % rpb)
    kg = jnp.zeros((B, S, M), i32).at[bix, pos].set(ki_s + bok).reshape(B, S * M)
    gcnt = (pc // 16).astype(i32)
    goff = (aoff[:, :nb] // 16).astype(i32)
    return qg, qsl, kg, gcnt, goff


def _sc_attention(Qp, Kp, Vp, qg, qsl, kg, gcnt, goff, B, Q, K, H, E, M):
    NC, NS = 2, 16
    dh = H // NHEADS
    RPT = Q // NS
    BPC = B // NC
    scale = 1.0 / math.sqrt(dh)
    ce = 1.0 / (E + 1e-6)
    S = qg.shape[1]

    mesh = plsc.VectorSubcoreMesh(core_axis_name="c", subcore_axis_name="s")

    @functools.partial(
        pl.kernel,
        mesh=mesh,
        compiler_params=pltpu.CompilerParams(
            needs_layout_passes=False, use_tc_tiling_on_sc=False),
        out_type=jax.ShapeDtypeStruct((B, Q, H), jnp.float32),
        scratch_types=[
            pltpu.VMEM((16,), jnp.int32),
            pltpu.VMEM((32,), jnp.int32),
            pltpu.VMEM((16 * M,), jnp.int32),
            pltpu.VMEM((32,), jnp.int32),
            pltpu.VMEM((32,), jnp.int32),
            pltpu.VMEM((16, H // 2), jnp.int32),
            pltpu.VMEM((16 * M, H // 2), jnp.int32),
            pltpu.VMEM((16 * M, H // 2), jnp.int32),
            pltpu.VMEM((RPT + 1, H), jnp.float32),
            pltpu.SemaphoreType.DMA,
            pltpu.SemaphoreType.DMA,
            pltpu.SemaphoreType.DMA,
        ],
    )
    def sc_kernel(qp, kp, vp, qg_h, qsl_h, kg_h, gcnt_h, goff_h, upd,
                  qgi_v, qsl_v, kgi_v, gcnt_v, goff_v,
                  qb, kb, vb, acc, s1, s2, s3):
        cid = lax.axis_index("c")
        sid = lax.axis_index("s")
        lane = jnp.arange(16, dtype=jnp.int32)

        def unpk(w):
            return plsc.unpack(plsc.bitcast(w, jnp.bfloat16),
                               format=plsc.PackFormat.INTERLEAVED)

        def compute_group(rows):
            for hp in range(NHEADS // 2):
                def sc_body(d, scs):
                    dv = jnp.bitwise_and(lane + d, dh - 1)
                    wv = dv + hp * dh
                    q0, q1 = unpk(plsc.load_gather(qb, [lane, wv]))
                    lo = []
                    hi = []
                    for m in range(M):
                        k0, k1 = unpk(plsc.load_gather(kb, [lane * M + m, wv]))
                        lo.append(scs[m] + q0 * k0)
                        hi.append(scs[M + m] + q1 * k1)
                    return tuple(lo + hi)

                scs = lax.fori_loop(
                    0, dh, sc_body,
                    tuple(jnp.zeros((16,), jnp.float32) for _ in range(2 * M)))

                ws = []
                for half in range(2):
                    ss = [scs[half * M + m] * scale for m in range(M)]
                    mx = ss[0]
                    for m in range(1, M):
                        mx = jnp.maximum(mx, ss[m])
                    es = [jnp.exp(s - mx) for s in ss]
                    tot = es[0]
                    for m in range(1, M):
                        tot = tot + es[m]
                    inv = ce / tot
                    ws.append([e * inv for e in es])

                def at_body(d, _c):
                    dv = jnp.bitwise_and(lane + d, dh - 1)
                    wv = dv + hp * dh
                    v0, v1 = unpk(plsc.load_gather(vb, [lane * M, wv]))
                    a0 = ws[0][0] * v0
                    a1 = ws[1][0] * v1
                    for m in range(1, M):
                        v0, v1 = unpk(plsc.load_gather(vb, [lane * M + m, wv]))
                        a0 = a0 + ws[0][m] * v0
                        a1 = a1 + ws[1][m] * v1
                    colb = dv * NHEADS + hp
                    plsc.addupdate_scatter(acc, [rows, colb], a0)
                    plsc.addupdate_scatter(acc, [rows, colb + NHEADS // 2], a1)
                    return _c

                lax.fori_loop(0, dh, at_body, 0)

        zero16 = jnp.zeros((16,), jnp.float32)

        for bb in range(BPC):
            b = cid * BPC + bb

            def zrow(r, c):
                for j in range(H // 16):
                    acc[r, pl.ds(j * 16, 16)] = zero16
                return c
            lax.fori_loop(0, RPT + 1, zrow, 0)

            pltpu.sync_copy(gcnt_h.at[b], gcnt_v.at[pl.ds(0, 16)])
            pltpu.sync_copy(goff_h.at[b], goff_v.at[pl.ds(0, 16)])
            ng = gcnt_v[pl.ds(sid, 16)][0]
            g0 = goff_v[pl.ds(sid, 16)][0]

            def group_body(g, c):
                sbase = (g0 + g) * 16
                pltpu.sync_copy(qg_h.at[b, pl.ds(sbase, 16)], qgi_v)
                pltpu.sync_copy(qsl_h.at[b, pl.ds(sbase, 16)],
                                qsl_v.at[pl.ds(0, 16)])
                pltpu.sync_copy(kg_h.at[b, pl.ds(sbase * M, 16 * M)], kgi_v)
                cq = pltpu.async_copy(qp.at[qgi_v], qb, s1)
                ck = pltpu.async_copy(kp.at[kgi_v], kb, s2)
                cv = pltpu.async_copy(vp.at[kgi_v], vb, s3)
                cq.wait()
                ck.wait()
                cv.wait()
                compute_group(qsl_v[pl.ds(0, 16)])
                return c

            lax.fori_loop(0, ng, group_body, 0)

            pltpu.sync_copy(acc.at[pl.ds(0, RPT)],
                            upd.at[b, pl.ds(sid * RPT, RPT)])

    return sc_kernel(Qp, Kp, Vp, qg, qsl, kg, gcnt, goff)


def kernel(q_nodes, k_nodes, hyperedges, Wq, bq, Wk, bk, Wv, bv,
           W1, b1, W2, b2, ln1_g, ln1_b, ln2_g, ln2_b):
    B, Q, H = q_nodes.shape
    K = k_nodes.shape[1]
    E = hyperedges.shape[1]
    M = hyperedges.shape[2] - 1

    qf = q_nodes.reshape(B * Q, H)
    kf = k_nodes.reshape(B * K, H)
    bf = jnp.bfloat16
    Qp = _proj_q(qf, Wq.T.astype(bf), bq)
    Kp, Vp = _proj_kv(kf, Wk.T.astype(bf), bk, Wv.T.astype(bf), bv)

    qi = jnp.minimum(hyperedges[:, :, 0], Q - 1).astype(jnp.int32)
    ki = jnp.minimum(hyperedges[:, :, 1:], K - 1).astype(jnp.int32)
    qg, qsl, kg, gcnt, goff = _build_routing(qi, ki, B, Q, K, E, M)

    upd = _sc_attention(Qp, Kp, Vp, qg, qsl, kg, gcnt, goff, B, Q, K, H, E, M)

    out = _post(qf, upd.reshape(B * Q, H), W1.T.astype(bf), b1,
                W2.T.astype(bf), b2, ln1_g, ln1_b, ln2_g, ln2_b)
    return out.reshape(B, Q, H)
